# baseline (speedup 1.0000x reference)
import jax
import jax.numpy as jnp
from jax import lax
from jax.experimental import pallas as pl
from jax.experimental.pallas import tpu as pltpu
from jax.experimental.pallas import tpu_sc as plsc

EMB_DIM = 32
BATCH = 16384
NROWS = 1000000
PACK = 8
SUPER = 128

NC = 2
NS = 16
NW = NC * NS
B_PER_W = BATCH // NW
CHUNK = 256
N_CHUNKS = B_PER_W // CHUNK

TP_BLK = 131072


TP_SUB = TP_BLK // PACK
TP_GRID = (NROWS + TP_BLK - 1) // TP_BLK
NLINES = TP_GRID * TP_SUB
T_LOG = TP_BLK.bit_length() - 1
S_LOG = TP_SUB.bit_length() - 1


def _tp_body(in_ref, o_ref):
    x = in_ref[...]
    lo = x[:EMB_DIM // 2, :]
    hi = x[EMB_DIM // 2:, :]
    lo_u = lax.convert_element_type(
        lax.bitcast_convert_type(lo.astype(jnp.bfloat16), jnp.uint16),
        jnp.uint32)
    hi_u = lax.convert_element_type(
        lax.bitcast_convert_type(hi.astype(jnp.bfloat16), jnp.uint16),
        jnp.uint32)
    packed = lax.bitcast_convert_type(lo_u | (hi_u << 16), jnp.int32)
    y = jnp.concatenate(
        [packed[:, a * TP_SUB:(a + 1) * TP_SUB] for a in range(PACK)], axis=0)
    o_ref[...] = jnp.transpose(y)


@jax.jit
def _tc_repack(tabT):
    return pl.pallas_call(
        _tp_body,
        grid=(TP_GRID,),
        in_specs=[pl.BlockSpec((EMB_DIM, TP_BLK), lambda i: (0, i))],
        out_specs=pl.BlockSpec((TP_SUB, SUPER), lambda i: (i, 0)),
        out_shape=jax.ShapeDtypeStruct((NLINES, SUPER), jnp.int32),
    )(tabT)


def _sc_extract_body(idx_hbm, tab_hbm, sel_hbm,
                     idx_v, q0_v, q1_v, t_v, sel_v, sem):
    wid = lax.axis_index("s") * NC + lax.axis_index("c")
    base = wid * B_PER_W
    pltpu.sync_copy(idx_hbm.at[pl.ds(base, B_PER_W)], idx_v)

    def to_line(r):
        return ((r >> T_LOG) << S_LOG) | (r & (TP_SUB - 1))

    def qbody(j, carry):
        q0_v[pl.ds(j * 16, 16)] = to_line(idx_v[pl.ds(j * 16, 16)])
        q1_v[pl.ds(j * 16, 16)] = to_line(idx_v[pl.ds(CHUNK + j * 16, 16)])
        return carry

    lax.fori_loop(0, CHUNK // 16, qbody, 0)

    iota16 = lax.iota(jnp.int32, 16)
    hi_mask = jnp.int32(-65536)

    def extract(c):
        def mbody(m, carry):
            rows = m * 16 + iota16
            r = idx_v[pl.ds(c * CHUNK + m * 16, 16)]
            off = ((r >> S_LOG) & (PACK - 1)) << 4
            for k in range(EMB_DIM // 2):
                w = plsc.load_gather(t_v, [rows, off + k])
                sel_v[k, pl.ds(c * CHUNK + m * 16, 16)] = plsc.bitcast(
                    w << 16, jnp.float32)
                sel_v[k + 16, pl.ds(c * CHUNK + m * 16, 16)] = plsc.bitcast(
                    w & hi_mask, jnp.float32)
            return carry

        lax.fori_loop(0, CHUNK // 16, mbody, 0)

    for c, q in enumerate((q0_v, q1_v)):
        pltpu.make_async_copy(tab_hbm.at[q], t_v, sem).start()
        pltpu.make_async_copy(tab_hbm.at[q], t_v, sem).wait()
        extract(c)

    pltpu.sync_copy(sel_v, sel_hbm.at[:, pl.ds(base, B_PER_W)])


def _sc_mul_body(idx_hbm, tab_hbm, usel_hbm, xt_hbm,
                 idx_v, q0_v, q1_v, t_v, usel_v, xt_v, sem):
    wid = lax.axis_index("s") * NC + lax.axis_index("c")
    base = wid * B_PER_W
    pltpu.sync_copy(idx_hbm.at[pl.ds(base, B_PER_W)], idx_v)
    pltpu.sync_copy(usel_hbm.at[:, pl.ds(base, B_PER_W)], usel_v)

    def to_line(r):
        return ((r >> T_LOG) << S_LOG) | (r & (TP_SUB - 1))

    def qbody(j, carry):
        q0_v[pl.ds(j * 16, 16)] = to_line(idx_v[pl.ds(j * 16, 16)])
        q1_v[pl.ds(j * 16, 16)] = to_line(idx_v[pl.ds(CHUNK + j * 16, 16)])
        return carry

    lax.fori_loop(0, CHUNK // 16, qbody, 0)

    iota16 = lax.iota(jnp.int32, 16)
    hi_mask = jnp.int32(-65536)

    def extract(c):
        def mbody(m, carry):
            rows = m * 16 + iota16
            pos = c * CHUNK + m * 16
            r = idx_v[pl.ds(pos, 16)]
            off = ((r >> S_LOG) & (PACK - 1)) << 4
            for k in range(EMB_DIM // 2):
                w = plsc.load_gather(t_v, [rows, off + k])
                vlo = plsc.bitcast(w << 16, jnp.float32)
                vhi = plsc.bitcast(w & hi_mask, jnp.float32)
                xt_v[k, pl.ds(pos, 16)] = usel_v[k, pl.ds(pos, 16)] * vlo
                xt_v[k + 16, pl.ds(pos, 16)] = (
                    usel_v[k + 16, pl.ds(pos, 16)] * vhi)
            return carry

        lax.fori_loop(0, CHUNK // 16, mbody, 0)

    for c, q in enumerate((q0_v, q1_v)):
        pltpu.make_async_copy(tab_hbm.at[q], t_v, sem).start()
        pltpu.make_async_copy(tab_hbm.at[q], t_v, sem).wait()
        extract(c)

    pltpu.sync_copy(xt_v, xt_hbm.at[:, pl.ds(base, B_PER_W)])


@jax.jit
def _sc_extract(idx, tab):
    mesh = plsc.VectorSubcoreMesh(core_axis_name="c", subcore_axis_name="s",
                                  num_cores=NC, num_subcores=NS)
    f = pl.kernel(
        _sc_extract_body,
        out_type=jax.ShapeDtypeStruct((EMB_DIM, BATCH), jnp.float32),
        mesh=mesh,
        scratch_types=[
            pltpu.VMEM((B_PER_W,), jnp.int32),
            pltpu.VMEM((CHUNK,), jnp.int32),
            pltpu.VMEM((CHUNK,), jnp.int32),
            pltpu.VMEM((CHUNK, SUPER), jnp.int32),
            pltpu.VMEM((EMB_DIM, B_PER_W), jnp.float32),
            pltpu.SemaphoreType.DMA,
        ],
        compiler_params=pltpu.CompilerParams(use_tc_tiling_on_sc=True,
                                             needs_layout_passes=False),
    )
    return f(idx, tab)


@jax.jit
def _sc_mul(idx, tab, usel):
    mesh = plsc.VectorSubcoreMesh(core_axis_name="c", subcore_axis_name="s",
                                  num_cores=NC, num_subcores=NS)
    f = pl.kernel(
        _sc_mul_body,
        out_type=jax.ShapeDtypeStruct((EMB_DIM, BATCH), jnp.float32),
        mesh=mesh,
        scratch_types=[
            pltpu.VMEM((B_PER_W,), jnp.int32),
            pltpu.VMEM((CHUNK,), jnp.int32),
            pltpu.VMEM((CHUNK,), jnp.int32),
            pltpu.VMEM((CHUNK, SUPER), jnp.int32),
            pltpu.VMEM((EMB_DIM, B_PER_W), jnp.float32),
            pltpu.VMEM((EMB_DIM, B_PER_W), jnp.float32),
            pltpu.SemaphoreType.DMA,
        ],
        compiler_params=pltpu.CompilerParams(use_tc_tiling_on_sc=True,
                                             needs_layout_passes=False),
    )
    return f(idx, tab, usel)


def _tc_mlp_body(xt_ref, w1_ref, b1_ref, w2_ref, b2_ref, o_ref):
    xt = xt_ref[...]
    h = jnp.dot(w1_ref[...], xt, preferred_element_type=jnp.float32)
    h = jnp.maximum(h + b1_ref[...], 0.0)
    logits = jnp.dot(w2_ref[...], h, preferred_element_type=jnp.float32)
    logits = logits + b2_ref[0, 0]
    o_ref[...] = 1.0 / (1.0 + jnp.exp(-logits))


@jax.jit
def _tc_mlp(xt, w1, b1, w2, b2):
    return pl.pallas_call(
        _tc_mlp_body,
        out_shape=jax.ShapeDtypeStruct((1, BATCH), jnp.float32),
    )(xt, w1, b1, w2, b2)


def kernel(user_idx, item_idx, user_table, item_table, W1, b1, W2, b2):
    utab_super = _tc_repack(user_table.T)
    usel = _sc_extract(user_idx, utab_super)
    itab_super = _tc_repack(item_table.T)
    xt = _sc_mul(item_idx, itab_super, usel)
    o = _tc_mlp(xt, W1, b1[:, None], W2, b2[None, :])
    return o[0]


# speedup vs baseline: 1.0108x; 1.0108x over previous
import jax
import jax.numpy as jnp
from jax import lax
from jax.experimental import pallas as pl
from jax.experimental.pallas import tpu as pltpu
from jax.experimental.pallas import tpu_sc as plsc

EMB_DIM = 32
BATCH = 16384
NROWS = 1000000
PACK = 8
SUPER = 128

NC = 2
NS = 16
NW = NC * NS
B_PER_W = BATCH // NW
CHUNK = 256
N_CHUNKS = B_PER_W // CHUNK

TP_BLK = 131072


TP_SUB = TP_BLK // PACK
TP_GRID = (NROWS + TP_BLK - 1) // TP_BLK
NLINES = TP_GRID * TP_SUB
T_LOG = TP_BLK.bit_length() - 1
S_LOG = TP_SUB.bit_length() - 1


def _tp_body(in_ref, o_ref):
    x = in_ref[...]
    lo = x[:EMB_DIM // 2, :]
    hi = x[EMB_DIM // 2:, :]
    lo_u = lax.convert_element_type(
        lax.bitcast_convert_type(lo.astype(jnp.bfloat16), jnp.uint16),
        jnp.uint32)
    hi_u = lax.convert_element_type(
        lax.bitcast_convert_type(hi.astype(jnp.bfloat16), jnp.uint16),
        jnp.uint32)
    packed = lax.bitcast_convert_type(lo_u | (hi_u << 16), jnp.int32)
    y = jnp.concatenate(
        [packed[:, a * TP_SUB:(a + 1) * TP_SUB] for a in range(PACK)], axis=0)
    o_ref[...] = jnp.transpose(y)


@jax.jit
def _tc_repack(tabT):
    return pl.pallas_call(
        _tp_body,
        grid=(TP_GRID,),
        in_specs=[pl.BlockSpec((EMB_DIM, TP_BLK), lambda i: (0, i))],
        out_specs=pl.BlockSpec((TP_SUB, SUPER), lambda i: (i, 0)),
        out_shape=jax.ShapeDtypeStruct((NLINES, SUPER), jnp.int32),
    )(tabT)


def _sc_body(uidx_hbm, iidx_hbm, utab_hbm, itab_hbm, xt_hbm,
             uidx_v, iidx_v, qu0_v, qu1_v, qi0_v, qi1_v,
             u_v, v_v, xt_v, sem_u, sem_i):
    wid = lax.axis_index("s") * NC + lax.axis_index("c")
    base = wid * B_PER_W
    pltpu.sync_copy(uidx_hbm.at[pl.ds(base, B_PER_W)], uidx_v)
    pltpu.sync_copy(iidx_hbm.at[pl.ds(base, B_PER_W)], iidx_v)

    def to_line(r):
        return ((r >> T_LOG) << S_LOG) | (r & (TP_SUB - 1))

    def qbody(j, carry):
        qu0_v[pl.ds(j * 16, 16)] = to_line(uidx_v[pl.ds(j * 16, 16)])
        qu1_v[pl.ds(j * 16, 16)] = to_line(uidx_v[pl.ds(CHUNK + j * 16, 16)])
        qi0_v[pl.ds(j * 16, 16)] = to_line(iidx_v[pl.ds(j * 16, 16)])
        qi1_v[pl.ds(j * 16, 16)] = to_line(iidx_v[pl.ds(CHUNK + j * 16, 16)])
        return carry

    lax.fori_loop(0, CHUNK // 16, qbody, 0)

    iota16 = lax.iota(jnp.int32, 16)

    hi_mask = jnp.int32(-65536)

    def extract(c):
        def mbody(m, carry):
            rows = m * 16 + iota16
            pos = c * CHUNK + m * 16
            iu = uidx_v[pl.ds(pos, 16)]
            ii = iidx_v[pl.ds(pos, 16)]
            off_u = ((iu >> S_LOG) & (PACK - 1)) << 4
            off_i = ((ii >> S_LOG) & (PACK - 1)) << 4
            for k in range(EMB_DIM // 2):
                uw = plsc.load_gather(u_v, [rows, off_u + k])
                vw = plsc.load_gather(v_v, [rows, off_i + k])
                ulo = plsc.bitcast(uw << 16, jnp.float32)
                vlo = plsc.bitcast(vw << 16, jnp.float32)
                uhi = plsc.bitcast(uw & hi_mask, jnp.float32)
                vhi = plsc.bitcast(vw & hi_mask, jnp.float32)
                xt_v[k, pl.ds(pos, 16)] = ulo * vlo
                xt_v[k + 16, pl.ds(pos, 16)] = uhi * vhi
            return carry

        lax.fori_loop(0, CHUNK // 16, mbody, 0)

    for c, (qu, qi) in enumerate(((qu0_v, qi0_v), (qu1_v, qi1_v))):
        cp_u = pltpu.make_async_copy(utab_hbm.at[qu], u_v, sem_u)
        cp_i = pltpu.make_async_copy(itab_hbm.at[qi], v_v, sem_i)
        cp_u.start()
        cp_i.start()
        cp_u.wait()
        cp_i.wait()
        extract(c)

    pltpu.sync_copy(xt_v, xt_hbm.at[:, pl.ds(base, B_PER_W)])


@jax.jit
def _sc_gather_mul(user_idx, item_idx, utab_super, itab_super):
    mesh = plsc.VectorSubcoreMesh(core_axis_name="c", subcore_axis_name="s",
                                  num_cores=NC, num_subcores=NS)
    f = pl.kernel(
        _sc_body,
        out_type=jax.ShapeDtypeStruct((EMB_DIM, BATCH), jnp.float32),
        mesh=mesh,
        scratch_types=[
            pltpu.VMEM((B_PER_W,), jnp.int32),
            pltpu.VMEM((B_PER_W,), jnp.int32),
            pltpu.VMEM((CHUNK,), jnp.int32),
            pltpu.VMEM((CHUNK,), jnp.int32),
            pltpu.VMEM((CHUNK,), jnp.int32),
            pltpu.VMEM((CHUNK,), jnp.int32),
            pltpu.VMEM((CHUNK, SUPER), jnp.int32),
            pltpu.VMEM((CHUNK, SUPER), jnp.int32),
            pltpu.VMEM((EMB_DIM, B_PER_W), jnp.float32),
            pltpu.SemaphoreType.DMA,
            pltpu.SemaphoreType.DMA,
        ],
        compiler_params=pltpu.CompilerParams(use_tc_tiling_on_sc=True,
                                             needs_layout_passes=False),
    )
    return f(user_idx, item_idx, utab_super, itab_super)


def _tc_mlp_body(xt_ref, w1_ref, b1_ref, w2_ref, b2_ref, o_ref):
    xt = xt_ref[...]
    h = jnp.dot(w1_ref[...], xt, preferred_element_type=jnp.float32)
    h = jnp.maximum(h + b1_ref[...], 0.0)
    logits = jnp.dot(w2_ref[...], h, preferred_element_type=jnp.float32)
    logits = logits + b2_ref[0, 0]
    o_ref[...] = 1.0 / (1.0 + jnp.exp(-logits))


@jax.jit
def _tc_mlp(xt, w1, b1, w2, b2):
    return pl.pallas_call(
        _tc_mlp_body,
        out_shape=jax.ShapeDtypeStruct((1, BATCH), jnp.float32),
    )(xt, w1, b1, w2, b2)


def kernel(user_idx, item_idx, user_table, item_table, W1, b1, W2, b2):
    utab_super = _tc_repack(user_table.T)
    itab_super = _tc_repack(item_table.T)
    xt = _sc_gather_mul(user_idx, item_idx, utab_super, itab_super)
    o = _tc_mlp(xt, W1, b1[:, None], W2, b2[None, :])
    return o[0]
